# baseline (speedup 1.0000x reference)
import functools

import jax
import jax.numpy as jnp
from jax import lax
from jax.experimental import pallas as pl
from jax.experimental.pallas import tpu as pltpu
from jax.experimental.pallas import tpu_sc as plsc

_NBLK = 1024
_T = 128
_TW = 128


def _knn_body(tT_ref, src_ref, sf_ref, idx_ref, w_ref, tab_ref):
    b = pl.program_id(0)
    j = pl.program_id(1)
    t = tT_ref[0]
    s4 = src_ref[0]
    m = s4.shape[0]
    nblk = t.shape[1]

    @pl.when(j == 0)
    def _():
        tab_ref[0, :, 0:64] = jnp.transpose(sf_ref[0], (1, 0))

    tt2 = jnp.sum(t * t, axis=0, keepdims=True)
    p = (jax.lax.dot(s4[:, :3], t, preferred_element_type=jnp.float32)
         + s4[:, 3:])

    big = jnp.float32(3e38)
    R, G = 8, 8
    bigrow = jnp.full((R, nblk), big, jnp.float32)
    a1 = [bigrow] * G
    a2 = [bigrow] * G
    a3 = [bigrow] * G
    for i in range(m // R):
        g = i % G
        row = p[i * R:(i + 1) * R, :]
        lo1 = jnp.minimum(a1[g], row)
        hi1 = jnp.maximum(a1[g], row)
        lo2 = jnp.minimum(a2[g], hi1)
        hi2 = jnp.maximum(a2[g], hi1)
        lo3 = jnp.minimum(a3[g], hi2)
        a1[g], a2[g], a3[g] = lo1, lo2, lo3

    cand = jnp.concatenate(a1 + a2 + a3, axis=0)
    v1 = jnp.min(cand, axis=0, keepdims=True)
    c2 = jnp.where(cand == v1, big, cand)
    v2 = jnp.min(c2, axis=0, keepdims=True)
    c3 = jnp.where(c2 == v2, big, c2)
    v3 = jnp.min(c3, axis=0, keepdims=True)

    iota = jax.lax.broadcasted_iota(jnp.int32, (m, nblk), 0)
    mfill = jnp.int32(m)
    am1 = jnp.min(jnp.where(p == v1, iota, mfill), axis=0, keepdims=True)
    am2 = jnp.min(jnp.where(p == v2, iota, mfill), axis=0, keepdims=True)
    am3 = jnp.min(jnp.where(p == v3, iota, mfill), axis=0, keepdims=True)

    rec1 = 1.0 / (jnp.sqrt(jnp.maximum(v1 + tt2, 1e-12)) + 1e-8)
    rec2 = 1.0 / (jnp.sqrt(jnp.maximum(v2 + tt2, 1e-12)) + 1e-8)
    rec3 = 1.0 / (jnp.sqrt(jnp.maximum(v3 + tt2, 1e-12)) + 1e-8)
    rnorm = 1.0 / (rec1 + rec2 + rec3)

    base = b * m
    last = mfill - 1
    am1 = jnp.minimum(am1, last)
    am2 = jnp.minimum(am2, last)
    am3 = jnp.minimum(am3, last)
    idx_ref[...] = jnp.concatenate([am1 + base, am2 + base, am3 + base], axis=0)
    w_ref[...] = jnp.concatenate([rec1 * rnorm, rec2 * rnorm, rec3 * rnorm],
                                 axis=0)


def _conv_body(ip_ref, tf_ref, w0_ref, y_ref, s_ref, ss_ref):
    b = pl.program_id(0)
    j = pl.program_id(1)
    ip = ip_ref[...]
    tf = tf_ref[0]
    w0 = w0_ref[...]
    yi = jax.lax.dot_general(w0[:, :64], ip, (((1,), (1,)), ((), ())),
                             preferred_element_type=jnp.float32)
    yt = jax.lax.dot(w0[:, 64:], tf, preferred_element_type=jnp.float32)
    y = yi + yt
    y_ref[0] = y

    @pl.when((b == 0) & (j == 0))
    def _():
        s_ref[...] = jnp.zeros_like(s_ref)
        ss_ref[...] = jnp.zeros_like(ss_ref)

    s_ref[...] += jnp.sum(y, axis=1, keepdims=True)
    ss_ref[...] += jnp.sum(y * y, axis=1, keepdims=True)


def _norm_body(y_ref, sc_ref, sh_ref, o_ref):
    y = y_ref[0]
    o_ref[0] = jnp.maximum(y * sc_ref[...] + sh_ref[...], 0.0)


def _make_sc_interp(BN, D):
    info = plsc.get_sparse_core_info()
    NC, NS = info.num_cores, info.num_subcores
    NW = NC * NS
    per_w = BN // NW
    nchunks = per_w // _T
    mesh = plsc.VectorSubcoreMesh(core_axis_name="c", subcore_axis_name="s")

    @functools.partial(
        pl.kernel, mesh=mesh,
        out_type=jax.ShapeDtypeStruct((BN, D), jnp.float32),
        scratch_types=(
            [pltpu.VMEM((_T,), jnp.int32)] * 6
            + [pltpu.VMEM((per_w,), jnp.float32)] * 3
            + [pltpu.VMEM((_T, _TW), jnp.float32)] * 6
            + [pltpu.VMEM((_T, D), jnp.float32),
               pltpu.SemaphoreType.DMA,
               pltpu.SemaphoreType.DMA]
        ),
    )
    def sc_interp(i0_hbm, i1_hbm, i2_hbm, w0_hbm, w1_hbm, w2_hbm,
                  tab_hbm, out_hbm,
                  ia0, ia1, ia2, ib0, ib1, ib2, w0v, w1v, w2v,
                  ga0, ga1, ga2, gb0, gb1, gb2, ob, semA, semB):
        wid = lax.axis_index("s") * NC + lax.axis_index("c")
        base = wid * per_w
        pltpu.sync_copy(w0_hbm.at[pl.ds(base, per_w)], w0v)
        pltpu.sync_copy(w1_hbm.at[pl.ds(base, per_w)], w1v)
        pltpu.sync_copy(w2_hbm.at[pl.ds(base, per_w)], w2v)

        ibufs = [(ia0, ia1, ia2), (ib0, ib1, ib2)]
        gbufs = [(ga0, ga1, ga2), (gb0, gb1, gb2)]
        sems = [semA, semB]

        def fire(c):
            par = c % 2
            i0, i1, i2 = ibufs[par]
            g0, g1, g2 = gbufs[par]
            off = base + c * _T
            pltpu.sync_copy(i0_hbm.at[pl.ds(off, _T)], i0)
            pltpu.sync_copy(i1_hbm.at[pl.ds(off, _T)], i1)
            pltpu.sync_copy(i2_hbm.at[pl.ds(off, _T)], i2)
            return (pltpu.async_copy(tab_hbm.at[i0], g0, sems[par]),
                    pltpu.async_copy(tab_hbm.at[i1], g1, sems[par]),
                    pltpu.async_copy(tab_hbm.at[i2], g2, sems[par]))

        pend = fire(0)
        for c in range(nchunks):
            nxt_pend = fire(c + 1) if c + 1 < nchunks else None
            for cp in pend:
                cp.wait()
            g0, g1, g2 = gbufs[c % 2]

            def gbody(g, carry):
                gbase = g * 16
                wv0g = w0v[pl.ds(c * _T + gbase, 16)]
                wv1g = w1v[pl.ds(c * _T + gbase, 16)]
                wv2g = w2v[pl.ds(c * _T + gbase, 16)]
                for t16 in range(16):
                    t = gbase + t16
                    w0s = wv0g[t16]
                    w1s = wv1g[t16]
                    w2s = wv2g[t16]
                    for f in range(D // 16):
                        sl = pl.ds(f * 16, 16)
                        ob[t, sl] = (g0[t, sl] * w0s + g1[t, sl] * w1s
                                     + g2[t, sl] * w2s)
                return carry

            lax.fori_loop(0, _T // 16, gbody, 0)
            pltpu.sync_copy(ob, out_hbm.at[pl.ds(base + c * _T, _T)])
            pend = nxt_pend

    return sc_interp


def kernel(target, source, target_feats, source_feats, W0, gamma0, beta0):
    B, n, _ = target.shape
    m = source.shape[1]
    C2 = source_feats.shape[1]
    nb = n // _NBLK
    BN = B * n
    tT = jnp.transpose(target, (0, 2, 1))
    src_aug = jnp.concatenate(
        [source * (-2.0), jnp.sum(source * source, -1, keepdims=True)], axis=-1)

    idx, wts, table = pl.pallas_call(
        _knn_body,
        grid=(B, nb),
        in_specs=[
            pl.BlockSpec((1, 3, _NBLK), lambda b, j: (b, 0, j)),
            pl.BlockSpec((1, m, 4), lambda b, j: (b, 0, 0)),
            pl.BlockSpec((1, C2, m), lambda b, j: (b, 0, 0)),
        ],
        out_specs=[
            pl.BlockSpec((3, _NBLK), lambda b, j: (0, b * (n // _NBLK) + j)),
            pl.BlockSpec((3, _NBLK), lambda b, j: (0, b * (n // _NBLK) + j)),
            pl.BlockSpec((1, m, _TW), lambda b, j: (b, 0, 0)),
        ],
        out_shape=[
            jax.ShapeDtypeStruct((3, BN), jnp.int32),
            jax.ShapeDtypeStruct((3, BN), jnp.float32),
            jax.ShapeDtypeStruct((B, m, _TW), jnp.float32),
        ],
        compiler_params=pltpu.CompilerParams(
            dimension_semantics=("arbitrary", "arbitrary")),
    )(tT, src_aug, source_feats)

    table_flat = table.reshape(B * m, _TW)
    interp = _make_sc_interp(BN, C2)(
        idx[0], idx[1], idx[2], wts[0], wts[1], wts[2], table_flat)

    y_raw, ssum, ssq = pl.pallas_call(
        _conv_body,
        grid=(B, nb),
        in_specs=[
            pl.BlockSpec((_NBLK, C2), lambda b, j: (b * (n // _NBLK) + j, 0)),
            pl.BlockSpec((1, 64, _NBLK), lambda b, j: (b, 0, j)),
            pl.BlockSpec((64, 128), lambda b, j: (0, 0)),
        ],
        out_specs=[
            pl.BlockSpec((1, 64, _NBLK), lambda b, j: (b, 0, j)),
            pl.BlockSpec((64, 1), lambda b, j: (0, 0)),
            pl.BlockSpec((64, 1), lambda b, j: (0, 0)),
        ],
        out_shape=[
            jax.ShapeDtypeStruct((B, 64, n), jnp.float32),
            jax.ShapeDtypeStruct((64, 1), jnp.float32),
            jax.ShapeDtypeStruct((64, 1), jnp.float32),
        ],
        compiler_params=pltpu.CompilerParams(
            dimension_semantics=("arbitrary", "arbitrary")),
    )(interp, target_feats, W0)

    cnt = jnp.float32(BN)
    mean = ssum[:, 0] / cnt
    var = ssq[:, 0] / cnt - mean * mean
    scale = gamma0 / jnp.sqrt(var + 1e-5)
    shift = beta0 - mean * scale

    out = pl.pallas_call(
        _norm_body,
        grid=(B, nb),
        in_specs=[
            pl.BlockSpec((1, 64, _NBLK), lambda b, j: (b, 0, j)),
            pl.BlockSpec((64, 1), lambda b, j: (0, 0)),
            pl.BlockSpec((64, 1), lambda b, j: (0, 0)),
        ],
        out_specs=pl.BlockSpec((1, 64, _NBLK), lambda b, j: (b, 0, j)),
        out_shape=jax.ShapeDtypeStruct((B, 64, n), jnp.float32),
        compiler_params=pltpu.CompilerParams(
            dimension_semantics=("parallel", "parallel")),
    )(y_raw, scale.reshape(64, 1), shift.reshape(64, 1))
    return out


# speedup vs baseline: 1.0089x; 1.0089x over previous
import functools

import jax
import jax.numpy as jnp
from jax import lax
from jax.experimental import pallas as pl
from jax.experimental.pallas import tpu as pltpu
from jax.experimental.pallas import tpu_sc as plsc

_NBLK = 1024
_T = 128
_TW = 128


def _knn_body(tT_ref, src_ref, sf_ref, idx_ref, w_ref, tab_ref):
    b = pl.program_id(0)
    j = pl.program_id(1)
    t = tT_ref[0]
    s4 = src_ref[0]
    m = s4.shape[0]
    nblk = t.shape[1]

    @pl.when(j == 0)
    def _():
        tab_ref[0, :, 0:64] = jnp.transpose(sf_ref[0], (1, 0))

    tt2 = jnp.sum(t * t, axis=0, keepdims=True)
    p = (jax.lax.dot(s4[:, :3], t, preferred_element_type=jnp.float32)
         + s4[:, 3:])

    big = jnp.float32(3e38)
    R, G = 8, 8
    bigrow = jnp.full((R, nblk), big, jnp.float32)
    a1 = [bigrow] * G
    a2 = [bigrow] * G
    a3 = [bigrow] * G
    for i in range(m // R):
        g = i % G
        row = p[i * R:(i + 1) * R, :]
        lo1 = jnp.minimum(a1[g], row)
        hi1 = jnp.maximum(a1[g], row)
        lo2 = jnp.minimum(a2[g], hi1)
        hi2 = jnp.maximum(a2[g], hi1)
        lo3 = jnp.minimum(a3[g], hi2)
        a1[g], a2[g], a3[g] = lo1, lo2, lo3

    cand = jnp.concatenate(a1 + a2 + a3, axis=0)
    v1 = jnp.min(cand, axis=0, keepdims=True)
    c2 = jnp.where(cand == v1, big, cand)
    v2 = jnp.min(c2, axis=0, keepdims=True)
    c3 = jnp.where(c2 == v2, big, c2)
    v3 = jnp.min(c3, axis=0, keepdims=True)

    iota = jax.lax.broadcasted_iota(jnp.int32, (m, nblk), 0)
    mfill = jnp.int32(m)
    am1 = jnp.min(jnp.where(p == v1, iota, mfill), axis=0, keepdims=True)
    am2 = jnp.min(jnp.where(p == v2, iota, mfill), axis=0, keepdims=True)
    am3 = jnp.min(jnp.where(p == v3, iota, mfill), axis=0, keepdims=True)

    rec1 = 1.0 / (jnp.sqrt(jnp.maximum(v1 + tt2, 1e-12)) + 1e-8)
    rec2 = 1.0 / (jnp.sqrt(jnp.maximum(v2 + tt2, 1e-12)) + 1e-8)
    rec3 = 1.0 / (jnp.sqrt(jnp.maximum(v3 + tt2, 1e-12)) + 1e-8)
    rnorm = 1.0 / (rec1 + rec2 + rec3)

    base = b * m
    last = mfill - 1
    am1 = jnp.minimum(am1, last)
    am2 = jnp.minimum(am2, last)
    am3 = jnp.minimum(am3, last)
    idx_ref[...] = jnp.concatenate([am1 + base, am2 + base, am3 + base], axis=0)
    w_ref[...] = jnp.concatenate([rec1 * rnorm, rec2 * rnorm, rec3 * rnorm],
                                 axis=0)


def _conv_body(ip_ref, tf_ref, w0_ref, y_ref, s_ref, ss_ref):
    b = pl.program_id(0)
    j = pl.program_id(1)
    ip = ip_ref[...]
    tf = tf_ref[0]
    w0 = w0_ref[...]
    yi = jax.lax.dot_general(w0[:, :64], ip, (((1,), (1,)), ((), ())),
                             preferred_element_type=jnp.float32)
    yt = jax.lax.dot(w0[:, 64:], tf, preferred_element_type=jnp.float32)
    y = yi + yt
    y_ref[0] = y

    @pl.when((b == 0) & (j == 0))
    def _():
        s_ref[...] = jnp.zeros_like(s_ref)
        ss_ref[...] = jnp.zeros_like(ss_ref)

    s_ref[...] += jnp.sum(y, axis=1, keepdims=True)
    ss_ref[...] += jnp.sum(y * y, axis=1, keepdims=True)


def _norm_body(y_ref, sc_ref, sh_ref, o_ref):
    y = y_ref[0]
    o_ref[0] = jnp.maximum(y * sc_ref[...] + sh_ref[...], 0.0)


def _make_sc_interp(BN, D):
    info = plsc.get_sparse_core_info()
    NC, NS = info.num_cores, info.num_subcores
    NW = NC * NS
    per_w = BN // NW
    nchunks = per_w // _T
    mesh = plsc.VectorSubcoreMesh(core_axis_name="c", subcore_axis_name="s")

    @functools.partial(
        pl.kernel, mesh=mesh,
        out_type=jax.ShapeDtypeStruct((BN, D), jnp.float32),
        scratch_types=(
            [pltpu.VMEM((_T,), jnp.int32)] * 6
            + [pltpu.VMEM((per_w,), jnp.float32)] * 3
            + [pltpu.VMEM((_T, _TW), jnp.float32)] * 6
            + [pltpu.VMEM((_T, D), jnp.float32),
               pltpu.SemaphoreType.DMA,
               pltpu.SemaphoreType.DMA]
        ),
    )
    def sc_interp(i0_hbm, i1_hbm, i2_hbm, w0_hbm, w1_hbm, w2_hbm,
                  tab_hbm, out_hbm,
                  ia0, ia1, ia2, ib0, ib1, ib2, w0v, w1v, w2v,
                  ga0, ga1, ga2, gb0, gb1, gb2, ob, semA, semB):
        wid = lax.axis_index("s") * NC + lax.axis_index("c")
        base = wid * per_w
        pltpu.sync_copy(w0_hbm.at[pl.ds(base, per_w)], w0v)
        pltpu.sync_copy(w1_hbm.at[pl.ds(base, per_w)], w1v)
        pltpu.sync_copy(w2_hbm.at[pl.ds(base, per_w)], w2v)

        ibufs = [(ia0, ia1, ia2), (ib0, ib1, ib2)]
        gbufs = [(ga0, ga1, ga2), (gb0, gb1, gb2)]
        sems = [semA, semB]

        def fire(c):
            par = c % 2
            i0, i1, i2 = ibufs[par]
            g0, g1, g2 = gbufs[par]
            off = base + c * _T
            pltpu.sync_copy(i0_hbm.at[pl.ds(off, _T)], i0)
            pltpu.sync_copy(i1_hbm.at[pl.ds(off, _T)], i1)
            pltpu.sync_copy(i2_hbm.at[pl.ds(off, _T)], i2)
            return (pltpu.async_copy(tab_hbm.at[i0], g0, sems[par]),
                    pltpu.async_copy(tab_hbm.at[i1], g1, sems[par]),
                    pltpu.async_copy(tab_hbm.at[i2], g2, sems[par]))

        pend = fire(0)
        for c in range(nchunks):
            nxt_pend = fire(c + 1) if c + 1 < nchunks else None
            for cp in pend:
                cp.wait()
            g0, g1, g2 = gbufs[c % 2]

            def gbody(g, carry):
                gbase = g * 16
                wv0g = w0v[pl.ds(c * _T + gbase, 16)]
                wv1g = w1v[pl.ds(c * _T + gbase, 16)]
                wv2g = w2v[pl.ds(c * _T + gbase, 16)]
                for t16 in range(16):
                    t = gbase + t16
                    w0s = wv0g[t16]
                    w1s = wv1g[t16]
                    w2s = wv2g[t16]
                    for f in range(D // 16):
                        sl = pl.ds(f * 16, 16)
                        ob[t, sl] = (g0[t, sl] * w0s + g1[t, sl] * w1s
                                     + g2[t, sl] * w2s)
                return carry

            lax.fori_loop(0, _T // 16, gbody, 0)
            pltpu.sync_copy(ob, out_hbm.at[pl.ds(base + c * _T, _T)])
            pend = nxt_pend

    return sc_interp


def kernel(target, source, target_feats, source_feats, W0, gamma0, beta0):
    B, n, _ = target.shape
    h = B // 2
    y0, s0, ss0 = _half(target[:h], source[:h], target_feats[:h],
                        source_feats[:h], W0)
    y1, s1, ss1 = _half(target[h:], source[h:], target_feats[h:],
                        source_feats[h:], W0)

    cnt = jnp.float32(B * n)
    mean = (s0[:, 0] + s1[:, 0]) / cnt
    var = (ss0[:, 0] + ss1[:, 0]) / cnt - mean * mean
    scale = (gamma0 / jnp.sqrt(var + 1e-5)).reshape(64, 1)
    shift = beta0.reshape(64, 1) - mean.reshape(64, 1) * scale

    o0 = _normalize(y0, scale, shift)
    o1 = _normalize(y1, scale, shift)
    return jnp.concatenate([o0, o1], axis=0)


def _normalize(y_raw, scale, shift):
    B, _, n = y_raw.shape
    nb = n // _NBLK
    return pl.pallas_call(
        _norm_body,
        grid=(B, nb),
        in_specs=[
            pl.BlockSpec((1, 64, _NBLK), lambda b, j: (b, 0, j)),
            pl.BlockSpec((64, 1), lambda b, j: (0, 0)),
            pl.BlockSpec((64, 1), lambda b, j: (0, 0)),
        ],
        out_specs=pl.BlockSpec((1, 64, _NBLK), lambda b, j: (b, 0, j)),
        out_shape=jax.ShapeDtypeStruct((B, 64, n), jnp.float32),
        compiler_params=pltpu.CompilerParams(
            dimension_semantics=("parallel", "parallel")),
    )(y_raw, scale, shift)


def _half(target, source, target_feats, source_feats, W0):
    B, n, _ = target.shape
    m = source.shape[1]
    C2 = source_feats.shape[1]
    nb = n // _NBLK
    BN = B * n
    tT = jnp.transpose(target, (0, 2, 1))
    src_aug = jnp.concatenate(
        [source * (-2.0), jnp.sum(source * source, -1, keepdims=True)], axis=-1)

    idx, wts, table = pl.pallas_call(
        _knn_body,
        grid=(B, nb),
        in_specs=[
            pl.BlockSpec((1, 3, _NBLK), lambda b, j: (b, 0, j)),
            pl.BlockSpec((1, m, 4), lambda b, j: (b, 0, 0)),
            pl.BlockSpec((1, C2, m), lambda b, j: (b, 0, 0)),
        ],
        out_specs=[
            pl.BlockSpec((3, _NBLK), lambda b, j: (0, b * (n // _NBLK) + j)),
            pl.BlockSpec((3, _NBLK), lambda b, j: (0, b * (n // _NBLK) + j)),
            pl.BlockSpec((1, m, _TW), lambda b, j: (b, 0, 0)),
        ],
        out_shape=[
            jax.ShapeDtypeStruct((3, BN), jnp.int32),
            jax.ShapeDtypeStruct((3, BN), jnp.float32),
            jax.ShapeDtypeStruct((B, m, _TW), jnp.float32),
        ],
        compiler_params=pltpu.CompilerParams(
            dimension_semantics=("arbitrary", "arbitrary")),
    )(tT, src_aug, source_feats)

    table_flat = table.reshape(B * m, _TW)
    interp = _make_sc_interp(BN, C2)(
        idx[0], idx[1], idx[2], wts[0], wts[1], wts[2], table_flat)

    y_raw, ssum, ssq = pl.pallas_call(
        _conv_body,
        grid=(B, nb),
        in_specs=[
            pl.BlockSpec((_NBLK, C2), lambda b, j: (b * (n // _NBLK) + j, 0)),
            pl.BlockSpec((1, 64, _NBLK), lambda b, j: (b, 0, j)),
            pl.BlockSpec((64, 128), lambda b, j: (0, 0)),
        ],
        out_specs=[
            pl.BlockSpec((1, 64, _NBLK), lambda b, j: (b, 0, j)),
            pl.BlockSpec((64, 1), lambda b, j: (0, 0)),
            pl.BlockSpec((64, 1), lambda b, j: (0, 0)),
        ],
        out_shape=[
            jax.ShapeDtypeStruct((B, 64, n), jnp.float32),
            jax.ShapeDtypeStruct((64, 1), jnp.float32),
            jax.ShapeDtypeStruct((64, 1), jnp.float32),
        ],
        compiler_params=pltpu.CompilerParams(
            dimension_semantics=("arbitrary", "arbitrary")),
    )(interp, target_feats, W0)

    return y_raw, ssum, ssq
